# Initial kernel scaffold; baseline (speedup 1.0000x reference)
import jax
import jax.numpy as jnp
from jax.experimental import pallas as pl


def kernel(char_code, char_len, token_code, spm_row, spm_val, char_emb):
    raise NotImplementedError("write your pallas kernel here")


# speedup vs baseline: 4.4534x; 4.4534x over previous
import jax
import jax.numpy as jnp
from jax import lax
from jax.experimental import pallas as pl
from jax.experimental.pallas import tpu as pltpu
from jax.experimental.pallas import tpu_sc as plsc

N_TOKENS = 50000
MAX_CHAR_LEN = 16
CHAR_VOCAB = 256
EMB_ROWS = CHAR_VOCAB + 16
D = 128
NNZ = 320000
N_MENTIONS = 10000

NW = 32
TPT = 1568
TOK_PAD = NW * TPT
TW = 112
NTW = TPT // TW

NNZ_PER = 10240
NNZ_PAD = NW * NNZ_PER
GW = 128
NGW = NNZ_PER // GW

_mesh = plsc.VectorSubcoreMesh(core_axis_name="c", subcore_axis_name="s")


def _wid():
    return lax.axis_index("s") * 2 + lax.axis_index("c")


def _encode_tokens(cc_packed, emb_pad):
    @pl.kernel(
        out_type=jax.ShapeDtypeStruct((TOK_PAD, D), jnp.float32),
        mesh=_mesh,
        scratch_types=[
            pltpu.VMEM((EMB_ROWS, D), jnp.float32),
            pltpu.VMEM((TW, MAX_CHAR_LEN), jnp.int32),
            pltpu.VMEM((TW, D), jnp.float32),
        ],
    )
    def enc(cc_hbm, emb_hbm, tf_hbm, emb_v, code_v, stage_v):
        wid = _wid()
        base = wid * TPT
        pltpu.sync_copy(emb_hbm, emb_v)

        def win_body(wi, _):
            off = base + wi * TW
            pltpu.sync_copy(cc_hbm.at[pl.ds(off, TW)], code_v)

            def tok_body(j, __):
                craw = code_v[j]
                ln = craw[0] >> 16
                cc = craw & 0xFFFF
                accs = [jnp.zeros((16,), jnp.float32) for _ in range(8)]
                for l in range(MAX_CHAR_LEN):
                    cid = jnp.where(l < ln, cc[l], CHAR_VOCAB)
                    for k in range(8):
                        accs[k] = accs[k] + emb_v[cid, pl.ds(16 * k, 16)]
                lnf = jnp.maximum(ln, 1).astype(jnp.float32)
                inv = 1.0 / jnp.full((16,), lnf, jnp.float32)
                for k in range(8):
                    stage_v[j, pl.ds(16 * k, 16)] = accs[k] * inv
                return 0

            lax.fori_loop(0, TW, tok_body, 0)
            pltpu.sync_copy(stage_v, tf_hbm.at[pl.ds(off, TW)])
            return 0

        lax.fori_loop(0, NTW, win_body, 0)

    return enc(cc_packed, emb_pad)


def _aggregate(tf, tc_p, sr_p, sv_p, zb):
    out_types = (
        jax.ShapeDtypeStruct((N_MENTIONS * D,), jnp.float32),
        jax.ShapeDtypeStruct((NW * 2 * D,), jnp.float32),
        jax.ShapeDtypeStruct((NW * 16,), jnp.int32),
    )

    @pl.kernel(
        out_type=out_types,
        mesh=_mesh,
        scratch_types=[
            pltpu.VMEM((NNZ_PER,), jnp.int32),
            pltpu.VMEM((NNZ_PER,), jnp.int32),
            pltpu.VMEM((NNZ_PER,), jnp.float32),
            pltpu.VMEM((16,), jnp.int32),
            pltpu.VMEM((GW, D), jnp.float32),
            pltpu.VMEM((32 * D,), jnp.float32),
            pltpu.VMEM((D,), jnp.float32),
            pltpu.VMEM((16,), jnp.int32),
            pltpu.SemaphoreType.DMA,
        ],
    )
    def agg(tf_hbm, tc_hbm, sr_hbm, sv_hbm, zb_hbm,
            out_hbm, side_hbm, srow_hbm,
            idx_v, row_v, val_v, zb_v, rows_v, zbuf, stage, srow_v, gsem):
        wid = _wid()
        base = wid * NNZ_PER
        pltpu.sync_copy(tc_hbm.at[pl.ds(base, NNZ_PER)], idx_v)
        pltpu.sync_copy(sr_hbm.at[pl.ds(base, NNZ_PER)], row_v)
        pltpu.sync_copy(sv_hbm.at[pl.ds(base, NNZ_PER)], val_v)
        pltpu.sync_copy(zb_hbm.at[wid], zb_v)

        zvec = jnp.zeros((16,), jnp.float32)
        for r in range(32 * 8):
            zbuf[pl.ds(16 * r, 16)] = zvec

        zbv = zb_v[...]
        zs = zbv[0]
        ze = zbv[1]
        n = ze - zs
        n32 = lax.div(n, 32)

        def z_body(k, _):
            pltpu.sync_copy(zbuf, out_hbm.at[pl.ds((zs + k * 32) * D, 32 * D)])
            return 0

        lax.fori_loop(0, n32, z_body, 0)
        rem_base = zs + n32 * 32

        def zr_body(k, _):
            pltpu.sync_copy(zbuf.at[pl.ds(0, D)],
                            out_hbm.at[pl.ds((rem_base + k) * D, D)])
            return 0

        lax.fori_loop(0, n - n32 * 32, zr_body, 0)

        r_first = row_v[pl.ds(0, 16)][0]
        r_last = row_v[pl.ds(NNZ_PER - 16, 16)][15]
        lanes = lax.broadcasted_iota(jnp.int32, (16,), 0)
        srvec = jnp.where(lanes == 0, r_first,
                          jnp.where(lanes == 1, r_last, 0))
        srow_v[...] = srvec
        pltpu.sync_copy(srow_v, srow_hbm.at[pl.ds(wid * 16, 16)])

        def flush_to(dst, accs):
            for k in range(8):
                stage[pl.ds(16 * k, 16)] = accs[k]
            pltpu.sync_copy(stage, dst)

        def win_body(w, carry):
            pltpu.async_copy(
                tf_hbm.at[idx_v.at[pl.ds(w * GW, GW)]], rows_v, gsem
            ).wait()

            def grp_body(g, c):
                jb = w * GW + g * 16
                rv = row_v[pl.ds(jb, 16)]
                vv = val_v[pl.ds(jb, 16)]
                cur_row, fc = c[0], c[1]
                accs = list(c[2:])
                for l in range(16):
                    r = rv[l]
                    v = vv[l]
                    p = r != cur_row

                    @pl.when(p & (fc == 0))
                    def _(accs=tuple(accs)):
                        flush_to(side_hbm.at[pl.ds(wid * 2 * D, D)], accs)

                    @pl.when(p & (fc > 0))
                    def _(cur_row=cur_row, accs=tuple(accs)):
                        flush_to(out_hbm.at[pl.ds(cur_row * D, D)], accs)

                    keep = (r == cur_row).astype(jnp.float32)
                    for k in range(8):
                        accs[k] = (v * rows_v[g * 16 + l, pl.ds(16 * k, 16)]
                                   + keep * accs[k])
                    fc = fc + p.astype(jnp.int32)
                    cur_row = r
                return (cur_row, fc) + tuple(accs)

            return lax.fori_loop(0, GW // 16, grp_body, carry)

        zero = jnp.zeros((16,), jnp.float32)
        init = (r_first, jnp.int32(0)) + (zero,) * 8
        fin = lax.fori_loop(0, NGW, win_body, init)
        fc = fin[1]
        accs = fin[2:]

        @pl.when(fc == 0)
        def _():
            flush_to(side_hbm.at[pl.ds(wid * 2 * D, D)], accs)
            pltpu.sync_copy(zbuf.at[pl.ds(0, D)],
                            side_hbm.at[pl.ds((wid * 2 + 1) * D, D)])

        @pl.when(fc > 0)
        def _():
            flush_to(side_hbm.at[pl.ds((wid * 2 + 1) * D, D)], accs)

    return agg(tf, tc_p, sr_p, sv_p, zb)


def _fixup(out_part, side, srows):
    def fix(out_in, side_ref, srow_ref, out_ref):
        out_ref[...] = out_in[...]
        for w in range(NW):
            for s in range(2):
                row = srow_ref[w * 16 + s]
                cur = out_ref[pl.ds(row, 1), :]
                out_ref[pl.ds(row, 1), :] = cur + side_ref[w, s, :].reshape(1, D)

    return pl.pallas_call(
        fix,
        out_shape=jax.ShapeDtypeStruct((N_MENTIONS, D), jnp.float32),
        in_specs=[
            pl.BlockSpec(memory_space=pltpu.VMEM),
            pl.BlockSpec(memory_space=pltpu.VMEM),
            pl.BlockSpec(memory_space=pltpu.SMEM),
        ],
        out_specs=pl.BlockSpec(memory_space=pltpu.VMEM),
    )(out_part, side, srows)


def kernel(char_code, char_len, token_code, spm_row, spm_val, char_emb):
    cc_p = jnp.pad(char_code, ((0, TOK_PAD - N_TOKENS), (0, 0)))
    cl_p = jnp.pad(char_len, (0, TOK_PAD - N_TOKENS))
    cc_packed = jnp.concatenate(
        [cc_p[:, :1] + (cl_p[:, None] << 16), cc_p[:, 1:]], axis=1
    )
    emb_pad = jnp.pad(char_emb, ((0, EMB_ROWS - CHAR_VOCAB), (0, 0)))

    npad = NNZ_PAD - NNZ
    tc_p = jnp.pad(token_code, (0, npad))
    sr_p = jnp.concatenate(
        [spm_row, jnp.full((npad,), spm_row[-1], jnp.int32)]
    )
    sv_p = jnp.pad(spm_val, (0, npad))
    bnd = sr_p[NNZ_PER::NNZ_PER]
    zs = jnp.concatenate([jnp.zeros((1,), jnp.int32), bnd])
    ze = jnp.concatenate([bnd, jnp.full((1,), N_MENTIONS, jnp.int32)])
    zb = jnp.concatenate([zs[:, None], ze[:, None],
                          jnp.zeros((NW, 14), jnp.int32)], axis=1)

    token_ft = _encode_tokens(cc_packed, emb_pad)
    out_part, side, srows = _aggregate(token_ft, tc_p, sr_p, sv_p, zb)
    return _fixup(out_part.reshape(N_MENTIONS, D),
                  side.reshape(NW, 2, D), srows)
